# baseline (speedup 1.0000x reference)
import jax
import jax.numpy as jnp
import numpy as np
from jax.experimental import pallas as pl
from jax.experimental.pallas import tpu as pltpu

_T = 256


def _bias_kernel(table_ref, out_ref, planes_ref, sems):
    h = pl.program_id(0)
    bi = pl.program_id(1)
    H = pl.num_programs(0)
    nb = out_ref.shape[3] // _T
    idx = h * nb + bi

    def _copy(hh, row, cj, kc):
        return pltpu.make_async_copy(
            planes_ref.at[kc],
            out_ref.at[0, hh, pl.ds(row * _T, _T), pl.ds(cj * _T, _T)],
            sems.at[cj],
        )

    @pl.when(idx > 0)
    def _wait_prev():
        prev = idx - 1
        ph = prev // nb
        pbi = prev % nb
        for cj in range(nb):
            pkc = jnp.clip(cj - pbi, -2, 2) + 2
            _copy(ph, pbi, cj, pkc).wait()

    @pl.when(bi == 0)
    def _build_planes():
        planes_ref[0] = jnp.full((_T, _T), table_ref[15, h], jnp.float32)
        planes_ref[4] = jnp.full((_T, _T), table_ref[31, h], jnp.float32)
        r = jax.lax.broadcasted_iota(jnp.int32, (_T, _T), 0)
        c = jax.lax.broadcasted_iota(jnp.int32, (_T, _T), 1)
        base = c - r
        for idx_p, koff in ((1, -_T), (2, 0), (3, _T)):
            d = base + koff
            rp = jnp.abs(d)
            is_small = rp < 8
            rp_safe = jnp.maximum(rp, 1).astype(jnp.float32)
            if_large = 8 + (
                jnp.log(rp_safe / 8) / np.log(128 / 8) * (16 - 8)
            ).astype(jnp.int32)
            if_large = jnp.minimum(if_large, 15)
            mag = jnp.where(is_small, rp, if_large)
            b = mag + jnp.where(d > 0, 16, 0)
            acc = jnp.full((_T, _T), table_ref[0, h], jnp.float32)
            for bb in range(1, 32):
                acc = jnp.where(b == bb, table_ref[bb, h], acc)
            planes_ref[idx_p] = acc

    for cj in range(nb):
        kc = jnp.clip(cj - bi, -2, 2) + 2
        _copy(h, bi, cj, kc).start()

    @pl.when(idx == H * nb - 1)
    def _final_wait():
        for cj in range(nb):
            kc = jnp.clip(cj - bi, -2, 2) + 2
            _copy(h, bi, cj, kc).wait()


def kernel(input_ids, bboxes, bias_table):
    B, S = input_ids.shape
    H = bias_table.shape[1]
    nb = S // _T
    out = pl.pallas_call(
        _bias_kernel,
        grid=(H, nb),
        in_specs=[pl.BlockSpec(memory_space=pltpu.SMEM)],
        out_specs=pl.BlockSpec(memory_space=pl.ANY),
        out_shape=jax.ShapeDtypeStruct((B, H, S, S), jnp.float32),
        scratch_shapes=[
            pltpu.VMEM((5, _T, _T), jnp.float32),
            pltpu.SemaphoreType.DMA((S // _T,)),
        ],
        compiler_params=pltpu.CompilerParams(
            dimension_semantics=("arbitrary", "arbitrary"),
        ),
    )(bias_table)
    return out


# speedup vs baseline: 2.3082x; 2.3082x over previous
import jax
import jax.numpy as jnp
import numpy as np
from jax.experimental import pallas as pl
from jax.experimental.pallas import tpu as pltpu

_T = 256


def _bias_kernel(table_ref, out_ref, mega_ref, sems):
    h = pl.program_id(0)
    bi = pl.program_id(1)
    H = pl.num_programs(0)
    S = out_ref.shape[3]
    nb = S // _T
    nm = nb + 7
    par = jax.lax.rem(h, 2)

    def _stripe_copy(hh, row, parity):
        return pltpu.make_async_copy(
            mega_ref.at[parity, :, pl.ds((nb - 1 - row) * _T, S)],
            out_ref.at[0, hh, pl.ds(row * _T, _T), :],
            sems.at[row],
        )

    @pl.when(h > 0)
    def _wait_prev():
        _stripe_copy(h - 1, bi, 1 - par).wait()

    @pl.when(bi == 0)
    def _build_mega():
        lo = nb - 2
        mega_ref[par, :, 0:lo * _T] = jnp.full(
            (_T, lo * _T), table_ref[15, h], jnp.float32)
        mega_ref[par, :, (lo + 3) * _T:nm * _T] = jnp.full(
            (_T, (nm - lo - 3) * _T), table_ref[31, h], jnp.float32)
        r = jax.lax.broadcasted_iota(jnp.int32, (_T, _T), 0)
        c = jax.lax.broadcasted_iota(jnp.int32, (_T, _T), 1)
        base = c - r
        for m, koff in ((lo, -_T), (lo + 1, 0), (lo + 2, _T)):
            d = base + koff
            rp = jnp.abs(d)
            is_small = rp < 8
            rp_safe = jnp.maximum(rp, 1).astype(jnp.float32)
            if_large = 8 + (
                jnp.log(rp_safe / 8) / np.log(128 / 8) * (16 - 8)
            ).astype(jnp.int32)
            if_large = jnp.minimum(if_large, 15)
            mag = jnp.where(is_small, rp, if_large)
            b = mag + jnp.where(d > 0, 16, 0)
            acc = jnp.full((_T, _T), table_ref[0, h], jnp.float32)
            for bb in range(1, 32):
                acc = jnp.where(b == bb, table_ref[bb, h], acc)
            mega_ref[par, :, m * _T:(m + 1) * _T] = acc

    _stripe_copy(h, bi, par).start()

    @pl.when(jnp.logical_and(h == H - 1, bi == nb - 1))
    def _final_wait():
        for row in range(nb):
            _stripe_copy(h, row, par).wait()


def kernel(input_ids, bboxes, bias_table):
    B, S = input_ids.shape
    H = bias_table.shape[1]
    nb = S // _T
    out = pl.pallas_call(
        _bias_kernel,
        grid=(H, nb),
        in_specs=[pl.BlockSpec(memory_space=pltpu.SMEM)],
        out_specs=pl.BlockSpec(memory_space=pl.ANY),
        out_shape=jax.ShapeDtypeStruct((B, H, S, S), jnp.float32),
        scratch_shapes=[
            pltpu.VMEM((2, _T, (nb + 7) * _T), jnp.float32),
            pltpu.SemaphoreType.DMA((nb,)),
        ],
        compiler_params=pltpu.CompilerParams(
            dimension_semantics=("arbitrary", "arbitrary"),
        ),
    )(bias_table)
    return out


# speedup vs baseline: 2.3196x; 1.0050x over previous
import jax
import jax.numpy as jnp
import numpy as np
from jax.experimental import pallas as pl
from jax.experimental.pallas import tpu as pltpu

_T = 256


def _bias_kernel(table_ref, out_ref, mega_ref, sems):
    h = pl.program_id(0)
    bi = pl.program_id(1)
    H = pl.num_programs(0)
    S = out_ref.shape[3]
    nb = S // _T
    nm = nb + 7
    par = jax.lax.rem(h, 3)

    def _stripe_copy(hh, row, parity):
        return pltpu.make_async_copy(
            mega_ref.at[parity, :, pl.ds((nb - 1 - row) * _T, S)],
            out_ref.at[0, hh, pl.ds(row * _T, _T), :],
            sems.at[jax.lax.rem(hh, 2), row],
        )

    @pl.when(h > 1)
    def _wait_prev():
        _stripe_copy(h - 2, bi, jax.lax.rem(h - 2, 3)).wait()

    @pl.when(bi == 0)
    def _build_mega():
        lo = nb - 2
        mega_ref[par, :, 0:lo * _T] = jnp.full(
            (_T, lo * _T), table_ref[15, h], jnp.float32)
        mega_ref[par, :, (lo + 3) * _T:nm * _T] = jnp.full(
            (_T, (nm - lo - 3) * _T), table_ref[31, h], jnp.float32)
        r = jax.lax.broadcasted_iota(jnp.int32, (_T, _T), 0)
        c = jax.lax.broadcasted_iota(jnp.int32, (_T, _T), 1)
        base = c - r
        for m, koff in ((lo, -_T), (lo + 1, 0), (lo + 2, _T)):
            d = base + koff
            rp = jnp.abs(d)
            is_small = rp < 8
            rp_safe = jnp.maximum(rp, 1).astype(jnp.float32)
            if_large = 8 + (
                jnp.log(rp_safe / 8) / np.log(128 / 8) * (16 - 8)
            ).astype(jnp.int32)
            if_large = jnp.minimum(if_large, 15)
            mag = jnp.where(is_small, rp, if_large)
            b = mag + jnp.where(d > 0, 16, 0)
            acc = jnp.full((_T, _T), table_ref[0, h], jnp.float32)
            for bb in range(1, 32):
                acc = jnp.where(b == bb, table_ref[bb, h], acc)
            mega_ref[par, :, m * _T:(m + 1) * _T] = acc

    _stripe_copy(h, bi, par).start()

    @pl.when(jnp.logical_and(h == H - 1, bi == nb - 1))
    def _final_wait():
        for row in range(nb):
            _stripe_copy(h - 1, row, jax.lax.rem(h - 1, 3)).wait()
        for row in range(nb):
            _stripe_copy(h, row, par).wait()


def kernel(input_ids, bboxes, bias_table):
    B, S = input_ids.shape
    H = bias_table.shape[1]
    nb = S // _T
    out = pl.pallas_call(
        _bias_kernel,
        grid=(H, nb),
        in_specs=[pl.BlockSpec(memory_space=pltpu.SMEM)],
        out_specs=pl.BlockSpec(memory_space=pl.ANY),
        out_shape=jax.ShapeDtypeStruct((B, H, S, S), jnp.float32),
        scratch_shapes=[
            pltpu.VMEM((3, _T, (nb + 7) * _T), jnp.float32),
            pltpu.SemaphoreType.DMA((2, nb)),
        ],
        compiler_params=pltpu.CompilerParams(
            dimension_semantics=("arbitrary", "arbitrary"),
        ),
    )(bias_table)
    return out
